# baseline (speedup 1.0000x reference)
import functools

import jax
import jax.numpy as jnp
from jax import lax
from jax.experimental import pallas as pl
from jax.experimental.pallas import tpu as pltpu
from jax.experimental.pallas import tpu_sc as plsc

F = 128
E = 320000
N = 10000
NPAD = 10240
NC, NS = 2, 16
EPT = E // NC // NS
B = 40
NB = EPT // B
ROWS_PT = NPAD // NS
INV_SQRT_DEG = 1.0 / (32.0 ** 0.5)

_MESH = dict(core_axis_name="c", subcore_axis_name="s",
             num_cores=NC, num_subcores=NS)


def _zero_fill(zbuf):
  def zrow(r, carry):
    for kk in range(F // 16):
      zbuf[r, pl.ds(kk * 16, 16)] = jnp.zeros((16,), jnp.float32)
    return carry
  lax.fori_loop(0, B, zrow, 0)


def _chunk_pipeline(idx_hbm, acc, base, rows, idxs, sls, sss, src_at):
  def load(b, p):
    e0 = base + b * B
    pltpu.async_copy(idx_hbm.at[pl.ds(e0, B)], idxs[p], sls[p])
    pltpu.async_copy(src_at(e0), rows[p], sls[p])

  def wait_load(p):
    pltpu.make_async_copy(idx_hbm.at[pl.ds(base, B)], idxs[p], sls[p]).wait()
    pltpu.make_async_copy(src_at(base), rows[p], sls[p]).wait()

  def scat(p):
    pltpu.async_copy(rows[p], acc.at[idxs[p]], sss[p], add=True)

  def wait_scat(p):
    pltpu.make_async_copy(rows[p], acc.at[idxs[p]], sss[p]).wait()

  for b in range(7):
    load(b, b % 9)
  wait_load(0)
  scat(0)
  load(7, 7)
  wait_load(1)
  scat(1)
  load(8, 8)

  def step(b, p, p2):
    wait_load(p)
    scat(p)
    wait_scat(p2)
    load(b + 7, p2)

  _PAR = ((2, 0), (3, 1), (4, 2), (5, 3), (6, 4), (7, 5), (8, 6), (0, 7),
          (1, 8))

  def body(g, carry):
    b0 = 9 * g + 2
    for i, (p, p2) in enumerate(_PAR):
      step(b0 + i, p, p2)
    return carry
  lax.fori_loop(0, (NB - 16) // 9, body, 0)

  for i, (p, p2) in enumerate(_PAR[:7]):
    step(NB - 14 + i, p, p2)
  for p in (0, 1, 2, 3, 4, 5, 6):
    wait_load(p)
    scat(p)
  for p in range(9):
    wait_scat(p)


def _sc_segment_sums(m0, m1p, idx):
  mesh = plsc.VectorSubcoreMesh(**_MESH)

  @functools.partial(
      pl.kernel,
      out_type=jax.ShapeDtypeStruct((4, NC, NPAD, F), jnp.float32),
      mesh=mesh,
      compiler_params=pltpu.CompilerParams(use_tc_tiling_on_sc=True),
      scratch_types=[
          pltpu.VMEM_SHARED((NPAD, F), jnp.float32),
          pltpu.VMEM((B, F), jnp.float32),
          pltpu.VMEM((B, F), jnp.float32),
          pltpu.VMEM((B, F), jnp.float32),
          pltpu.VMEM((B, F), jnp.float32),
          pltpu.VMEM((B, F), jnp.float32),
          pltpu.VMEM((B, F), jnp.float32),
          pltpu.VMEM((B, F), jnp.float32),
          pltpu.VMEM((B, F), jnp.float32),
          pltpu.VMEM((B, F), jnp.float32),
          pltpu.VMEM((B,), jnp.int32),
          pltpu.VMEM((B,), jnp.int32),
          pltpu.VMEM((B,), jnp.int32),
          pltpu.VMEM((B,), jnp.int32),
          pltpu.VMEM((B,), jnp.int32),
          pltpu.VMEM((B,), jnp.int32),
          pltpu.VMEM((B,), jnp.int32),
          pltpu.VMEM((B,), jnp.int32),
          pltpu.VMEM((B,), jnp.int32),
          pltpu.SemaphoreType.DMA,
          pltpu.SemaphoreType.DMA,
          pltpu.SemaphoreType.DMA,
          pltpu.SemaphoreType.DMA,
          pltpu.SemaphoreType.DMA,
          pltpu.SemaphoreType.DMA,
          pltpu.SemaphoreType.DMA,
          pltpu.SemaphoreType.DMA,
          pltpu.SemaphoreType.DMA,
          pltpu.SemaphoreType.DMA,
          pltpu.SemaphoreType.DMA,
          pltpu.SemaphoreType.DMA,
          pltpu.SemaphoreType.DMA,
          pltpu.SemaphoreType.DMA,
          pltpu.SemaphoreType.DMA,
          pltpu.SemaphoreType.DMA,
          pltpu.SemaphoreType.DMA,
          pltpu.SemaphoreType.DMA,
      ],
  )
  def k(m0_hbm, m1p_hbm, idx_hbm, out_hbm, acc, *bufs):
    rows = bufs[0:9]
    idxs = bufs[9:18]
    sls = bufs[18:27]
    sss = bufs[27:36]
    s = lax.axis_index("s")
    c = lax.axis_index("c")
    base = c * (E // NC) + s * EPT

    def run(src_at, plane):
      _zero_fill(rows[0])
      zcopies = []
      for z in range(ROWS_PT // B):
        zcopies.append(pltpu.async_copy(
            rows[0], acc.at[pl.ds(s * ROWS_PT + z * B, B)], sls[z % 9]))
      for zc in zcopies:
        zc.wait()
      plsc.subcore_barrier()
      _chunk_pipeline(idx_hbm, acc, base, rows, idxs, sls, sss, src_at)
      plsc.subcore_barrier()
      pltpu.sync_copy(acc.at[pl.ds(s * ROWS_PT, ROWS_PT)],
                      out_hbm.at[plane, c, pl.ds(s * ROWS_PT, ROWS_PT)])

    run(lambda e0: m0_hbm.at[pl.ds(e0, B)], 0)
    for j in range(3):
      run(lambda e0, j=j: m1p_hbm.at[j, pl.ds(e0, B)], 1 + j)

  return k(m0, m1p, idx)


BLK = 1000


def _tc_body(den, w1a, w1b, b1r, w2, b2r, lw, s1, sb1r, s2, sb2r,
             dh0, dh1):
  cs = INV_SQRT_DEG
  x = den[...]
  den0 = (x[0, 0] + x[0, 1]) * cs
  a = (x[1, 0] + x[1, 1]) * cs
  b = (x[2, 0] + x[2, 1]) * cs
  d = (x[3, 0] + x[3, 1]) * cs
  inv1 = jnp.sqrt(a * a + b * b + d * d + 1e-8)
  f32 = jnp.float32
  def dott(x, w):
    return lax.dot_general(x, w[...], (((1,), (1,)), ((), ())),
                           preferred_element_type=f32)
  h = dott(den0, w1a) + dott(inv1, w1b) + b1r[...]
  h = h * jax.nn.sigmoid(h)
  dh0v = dott(h, w2) + b2r[...]
  sh = dott(dh0v, s1) + sb1r[...]
  sh = sh * jax.nn.sigmoid(sh)
  alpha = dott(sh, s2) + sb2r[...]
  dh0[...] = dh0v
  dh1[0] = dott(a, lw) * alpha
  dh1[1] = dott(b, lw) * alpha
  dh1[2] = dott(d, lw) * alpha


def _tc_mlp(den, w1a, w1b, b1, w2, b2, lw, s1, sb1, s2, sb2):
  wspec = lambda shape: pl.BlockSpec(shape, lambda i: (0,) * len(shape))
  return pl.pallas_call(
      _tc_body,
      grid=(N // BLK,),
      in_specs=[
          pl.BlockSpec((4, NC, BLK, F), lambda i: (0, 0, i, 0)),
          wspec((F, F)), wspec((F, F)), wspec((1, F)),
          wspec((F, F)), wspec((1, F)), wspec((F, F)),
          wspec((F, F)), wspec((1, F)), wspec((F, F)), wspec((1, F)),
      ],
      out_specs=[
          pl.BlockSpec((BLK, F), lambda i: (i, 0)),
          pl.BlockSpec((3, BLK, F), lambda i: (0, i, 0)),
      ],
      out_shape=[
          jax.ShapeDtypeStruct((N, F), jnp.float32),
          jax.ShapeDtypeStruct((3, N, F), jnp.float32),
      ],
      compiler_params=pltpu.CompilerParams(
          dimension_semantics=("arbitrary",)),
  )(den, w1a, w1b, b1, w2, b2, lw, s1, sb1, s2, sb2)


def kernel(msgs_0, msgs_1, index, num_nodes, W1, b1, W2, b2, L1W, S1, sb1, S2,
           sb2):
  idxc = jnp.minimum(index, num_nodes - 1).astype(jnp.int32)
  m1p = jnp.transpose(msgs_1, (1, 0, 2))
  den = _sc_segment_sums(msgs_0, m1p, idxc)
  dh0, dh1p = _tc_mlp(
      den,
      W1[:, :F], W1[:, F:], b1.reshape(1, F),
      W2, b2.reshape(1, F), L1W,
      S1, sb1.reshape(1, F), S2, sb2.reshape(1, F))
  dh1 = jnp.transpose(dh1p, (1, 0, 2))
  return (dh0, dh1)


# speedup vs baseline: 1.0039x; 1.0039x over previous
import functools

import jax
import jax.numpy as jnp
from jax import lax
from jax.experimental import pallas as pl
from jax.experimental.pallas import tpu as pltpu
from jax.experimental.pallas import tpu_sc as plsc

F = 128
E = 320000
N = 10000
NPAD = 10240
NC, NS = 2, 16
EPT = E // NC // NS
B = 40
NB = EPT // B
ROWS_PT = NPAD // NS
INV_SQRT_DEG = 1.0 / (32.0 ** 0.5)

_MESH = dict(core_axis_name="c", subcore_axis_name="s",
             num_cores=NC, num_subcores=NS)


def _zero_fill(zbuf):
  def zrow(r, carry):
    for kk in range(F // 16):
      zbuf[r, pl.ds(kk * 16, 16)] = jnp.zeros((16,), jnp.float32)
    return carry
  lax.fori_loop(0, B, zrow, 0)


def _chunk_pipeline(idx_hbm, acc, base, rows, idxs, sls, sss, src_at):
  def load(b, p):
    e0 = base + b * B
    pltpu.async_copy(idx_hbm.at[pl.ds(e0, B)], idxs[p], sls[p])
    pltpu.async_copy(src_at(e0), rows[p], sls[p])

  def wait_load(p):
    pltpu.make_async_copy(idx_hbm.at[pl.ds(base, B)], idxs[p], sls[p]).wait()
    pltpu.make_async_copy(src_at(base), rows[p], sls[p]).wait()

  def scat(p):
    pltpu.async_copy(rows[p], acc.at[idxs[p]], sss[p], add=True)

  def wait_scat(p):
    pltpu.make_async_copy(rows[p], acc.at[idxs[p]], sss[p]).wait()

  for b in range(6):
    load(b, b % 8)
  wait_load(0)
  scat(0)
  load(6, 6)
  wait_load(1)
  scat(1)
  load(7, 7)

  def step(b, p, p2):
    wait_load(p)
    scat(p)
    wait_scat(p2)
    load(b + 6, p2)

  def body(g, carry):
    b0 = 8 * g + 2
    for i, (p, p2) in enumerate(
        ((2, 0), (3, 1), (4, 2), (5, 3), (6, 4), (7, 5), (0, 6), (1, 7))):
      step(b0 + i, p, p2)
    return carry
  lax.fori_loop(0, (NB - 10) // 8, body, 0)

  step(NB - 8, 2, 0)
  step(NB - 7, 3, 1)
  for p in (4, 5, 6, 7, 0, 1):
    wait_load(p)
    scat(p)
  for p in range(8):
    wait_scat(p)


def _sc_segment_sums(m0, m1p, idx):
  mesh = plsc.VectorSubcoreMesh(**_MESH)

  @functools.partial(
      pl.kernel,
      out_type=jax.ShapeDtypeStruct((4, NC, NPAD, F), jnp.float32),
      mesh=mesh,
      compiler_params=pltpu.CompilerParams(use_tc_tiling_on_sc=True),
      scratch_types=[
          pltpu.VMEM_SHARED((NPAD, F), jnp.float32),
          pltpu.VMEM((B, F), jnp.float32),
          pltpu.VMEM((B, F), jnp.float32),
          pltpu.VMEM((B, F), jnp.float32),
          pltpu.VMEM((B, F), jnp.float32),
          pltpu.VMEM((B, F), jnp.float32),
          pltpu.VMEM((B, F), jnp.float32),
          pltpu.VMEM((B, F), jnp.float32),
          pltpu.VMEM((B, F), jnp.float32),
          pltpu.VMEM((B,), jnp.int32),
          pltpu.VMEM((B,), jnp.int32),
          pltpu.VMEM((B,), jnp.int32),
          pltpu.VMEM((B,), jnp.int32),
          pltpu.VMEM((B,), jnp.int32),
          pltpu.VMEM((B,), jnp.int32),
          pltpu.VMEM((B,), jnp.int32),
          pltpu.VMEM((B,), jnp.int32),
          pltpu.SemaphoreType.DMA,
          pltpu.SemaphoreType.DMA,
          pltpu.SemaphoreType.DMA,
          pltpu.SemaphoreType.DMA,
          pltpu.SemaphoreType.DMA,
          pltpu.SemaphoreType.DMA,
          pltpu.SemaphoreType.DMA,
          pltpu.SemaphoreType.DMA,
          pltpu.SemaphoreType.DMA,
          pltpu.SemaphoreType.DMA,
          pltpu.SemaphoreType.DMA,
          pltpu.SemaphoreType.DMA,
          pltpu.SemaphoreType.DMA,
          pltpu.SemaphoreType.DMA,
          pltpu.SemaphoreType.DMA,
          pltpu.SemaphoreType.DMA,
      ],
  )
  def k(m0_hbm, m1p_hbm, idx_hbm, out_hbm, acc, *bufs):
    rows = bufs[0:8]
    idxs = bufs[8:16]
    sls = bufs[16:24]
    sss = bufs[24:32]
    s = lax.axis_index("s")
    c = lax.axis_index("c")
    base = c * (E // NC) + s * EPT

    def run(src_at, plane):
      _zero_fill(rows[0])
      zcopies = []
      for z in range(ROWS_PT // B):
        zcopies.append(pltpu.async_copy(
            rows[0], acc.at[pl.ds(s * ROWS_PT + z * B, B)], sls[z % 8]))
      for zc in zcopies:
        zc.wait()
      plsc.subcore_barrier()
      _chunk_pipeline(idx_hbm, acc, base, rows, idxs, sls, sss, src_at)
      plsc.subcore_barrier()
      pltpu.sync_copy(acc.at[pl.ds(s * ROWS_PT, ROWS_PT)],
                      out_hbm.at[plane, c, pl.ds(s * ROWS_PT, ROWS_PT)])

    run(lambda e0: m0_hbm.at[pl.ds(e0, B)], 0)
    for j in range(3):
      run(lambda e0, j=j: m1p_hbm.at[j, pl.ds(e0, B)], 1 + j)

  return k(m0, m1p, idx)


BLK = 1000


def _tc_body(den, w1a, w1b, b1r, w2, b2r, lw, s1, sb1r, s2, sb2r,
             dh0, dh1):
  cs = INV_SQRT_DEG
  x = den[...]
  den0 = (x[0, 0] + x[0, 1]) * cs
  a = (x[1, 0] + x[1, 1]) * cs
  b = (x[2, 0] + x[2, 1]) * cs
  d = (x[3, 0] + x[3, 1]) * cs
  inv1 = jnp.sqrt(a * a + b * b + d * d + 1e-8)
  f32 = jnp.float32
  h = (jnp.dot(den0, w1a[...], preferred_element_type=f32)
       + jnp.dot(inv1, w1b[...], preferred_element_type=f32) + b1r[...])
  h = h * jax.nn.sigmoid(h)
  dh0v = jnp.dot(h, w2[...], preferred_element_type=f32) + b2r[...]
  sh = jnp.dot(dh0v, s1[...], preferred_element_type=f32) + sb1r[...]
  sh = sh * jax.nn.sigmoid(sh)
  alpha = jnp.dot(sh, s2[...], preferred_element_type=f32) + sb2r[...]
  dh0[...] = dh0v
  dh1[0] = jnp.dot(a, lw[...], preferred_element_type=f32) * alpha
  dh1[1] = jnp.dot(b, lw[...], preferred_element_type=f32) * alpha
  dh1[2] = jnp.dot(d, lw[...], preferred_element_type=f32) * alpha


def _tc_mlp(den, w1a, w1b, b1, w2, b2, lw, s1, sb1, s2, sb2):
  wspec = lambda shape: pl.BlockSpec(shape, lambda i: (0,) * len(shape))
  return pl.pallas_call(
      _tc_body,
      grid=(N // BLK,),
      in_specs=[
          pl.BlockSpec((4, NC, BLK, F), lambda i: (0, 0, i, 0)),
          wspec((F, F)), wspec((F, F)), wspec((1, F)),
          wspec((F, F)), wspec((1, F)), wspec((F, F)),
          wspec((F, F)), wspec((1, F)), wspec((F, F)), wspec((1, F)),
      ],
      out_specs=[
          pl.BlockSpec((BLK, F), lambda i: (i, 0)),
          pl.BlockSpec((3, BLK, F), lambda i: (0, i, 0)),
      ],
      out_shape=[
          jax.ShapeDtypeStruct((N, F), jnp.float32),
          jax.ShapeDtypeStruct((3, N, F), jnp.float32),
      ],
      compiler_params=pltpu.CompilerParams(
          dimension_semantics=("arbitrary",)),
  )(den, w1a, w1b, b1, w2, b2, lw, s1, sb1, s2, sb2)


def kernel(msgs_0, msgs_1, index, num_nodes, W1, b1, W2, b2, L1W, S1, sb1, S2,
           sb2):
  idxc = jnp.minimum(index, num_nodes - 1).astype(jnp.int32)
  m1p = jnp.transpose(msgs_1, (1, 0, 2))
  den = _sc_segment_sums(msgs_0, m1p, idxc)
  w1t = W1.T
  dh0, dh1p = _tc_mlp(
      den,
      w1t[:F], w1t[F:], b1.reshape(1, F),
      W2.T, b2.reshape(1, F), L1W.T,
      S1.T, sb1.reshape(1, F), S2.T, sb2.reshape(1, F))
  dh1 = jnp.transpose(dh1p, (1, 0, 2))
  return (dh0, dh1)
